# Initial kernel scaffold; baseline (speedup 1.0000x reference)
import jax
import jax.numpy as jnp
from jax.experimental import pallas as pl


def kernel(x, edge_index, W1, W2):
    raise NotImplementedError("write your pallas kernel here")


# speedup vs baseline: 8.3008x; 8.3008x over previous
import functools

import jax
import jax.numpy as jnp
from jax import lax
from jax.experimental import pallas as pl
from jax.experimental.pallas import tpu as pltpu
from jax.experimental.pallas import tpu_sc as plsc

NC = 2
NS = 16
CK = 128


def _make_deg_kernel(N, E):
    EW = E // (NC * NS)
    NFULL = EW // CK
    REM = EW - NFULL * CK
    ZR = 80
    Q = ((N + NS - 1) // NS + ZR - 1) // ZR * ZR
    assert N % ZR == 0
    mesh = plsc.VectorSubcoreMesh(core_axis_name="c", subcore_axis_name="s")

    @functools.partial(
        pl.kernel,
        out_type=jax.ShapeDtypeStruct((2, NC, N, 16), jnp.float32),
        mesh=mesh,
        scratch_types=[
            pltpu.VMEM((CK,), jnp.int32),
            pltpu.VMEM((CK, 16), jnp.float32),
            pltpu.VMEM((ZR, 16), jnp.float32),
            pltpu.VMEM_SHARED((N, 16), jnp.float32),
            pltpu.VMEM_SHARED((N, 16), jnp.float32),
        ],
    )
    def deg(src_hbm, dst_hbm, out_hbm, idx_v, ones_v, zbuf_v, tsrc, tdst):
        cid = lax.axis_index("c")
        sid = lax.axis_index("s")
        w = cid * NS + sid
        r0 = sid * Q
        nk = (jnp.minimum(r0 + Q, N) - jnp.minimum(r0, N)) // ZR

        def fill(i, _):
            ones_v[i, :] = jnp.ones((16,), jnp.float32)
            return 0

        lax.fori_loop(0, CK, fill, 0)

        def zfill(i, _):
            zbuf_v[i, :] = jnp.zeros((16,), jnp.float32)
            return 0

        lax.fori_loop(0, ZR, zfill, 0)

        def zcopy(k, _):
            pltpu.sync_copy(zbuf_v, tsrc.at[pl.ds(r0 + k * ZR, ZR)])
            pltpu.sync_copy(zbuf_v, tdst.at[pl.ds(r0 + k * ZR, ZR)])
            return 0

        lax.fori_loop(0, nk, zcopy, 0)
        plsc.subcore_barrier()

        base = w * EW

        def body(i, _):
            off = base + i * CK
            pltpu.sync_copy(src_hbm.at[pl.ds(off, CK)], idx_v)
            pltpu.sync_copy(ones_v, tsrc.at[idx_v], add=True)
            pltpu.sync_copy(dst_hbm.at[pl.ds(off, CK)], idx_v)
            pltpu.sync_copy(ones_v, tdst.at[idx_v], add=True)
            return 0

        lax.fori_loop(0, NFULL, body, 0)
        if REM:
            off = base + NFULL * CK
            ridx = idx_v.at[pl.ds(0, REM)]
            pltpu.sync_copy(src_hbm.at[pl.ds(off, REM)], ridx)
            pltpu.sync_copy(ones_v.at[pl.ds(0, REM)], tsrc.at[ridx], add=True)
            pltpu.sync_copy(dst_hbm.at[pl.ds(off, REM)], ridx)
            pltpu.sync_copy(ones_v.at[pl.ds(0, REM)], tdst.at[ridx], add=True)
        plsc.subcore_barrier()

        def outcopy(k, _):
            rk = r0 + k * ZR
            pltpu.sync_copy(tsrc.at[pl.ds(rk, ZR)], zbuf_v)
            pltpu.sync_copy(zbuf_v, out_hbm.at[0, cid, pl.ds(rk, ZR)])
            pltpu.sync_copy(tdst.at[pl.ds(rk, ZR)], zbuf_v)
            pltpu.sync_copy(zbuf_v, out_hbm.at[1, cid, pl.ds(rk, ZR)])
            return 0

        lax.fori_loop(0, nk, outcopy, 0)

    return deg


def _make_prop_kernel(N, E, D):
    EW = E // (NC * NS)
    NFULL = EW // CK
    REM = EW - NFULL * CK
    ZR = 80
    Q = ((N + NS - 1) // NS + ZR - 1) // ZR * ZR
    assert N % ZR == 0
    mesh = plsc.VectorSubcoreMesh(core_axis_name="c", subcore_axis_name="s")

    @functools.partial(
        pl.kernel,
        out_type=jax.ShapeDtypeStruct((NC, N, D), jnp.float32),
        mesh=mesh,
        scratch_types=[
            pltpu.VMEM((CK,), jnp.int32),
            pltpu.VMEM((CK,), jnp.int32),
            pltpu.VMEM((CK, D), jnp.float32),
            pltpu.VMEM((ZR, D), jnp.float32),
            pltpu.VMEM_SHARED((N, D), jnp.float32),
            pltpu.SemaphoreType.DMA,
        ],
        compiler_params=pltpu.CompilerParams(use_tc_tiling_on_sc=False),
    )
    def prop(p_hbm, src_hbm, dst_hbm, out_hbm, sidx, didx, rows, zbuf, acc, sem):
        cid = lax.axis_index("c")
        sid = lax.axis_index("s")
        w = cid * NS + sid
        ng = D // 16
        r0 = sid * Q
        nk = (jnp.minimum(r0 + Q, N) - jnp.minimum(r0, N)) // ZR

        def zfill(i, _):
            for g in range(ng):
                zbuf[i, pl.ds(g * 16, 16)] = jnp.zeros((16,), jnp.float32)
            return 0

        lax.fori_loop(0, ZR, zfill, 0)

        def zcopy(k, _):
            pltpu.sync_copy(zbuf, acc.at[pl.ds(r0 + k * ZR, ZR)])
            return 0

        lax.fori_loop(0, nk, zcopy, 0)
        plsc.subcore_barrier()

        base = w * EW

        def body(i, _):
            off = base + i * CK
            pltpu.sync_copy(src_hbm.at[pl.ds(off, CK)], sidx)
            pltpu.sync_copy(dst_hbm.at[pl.ds(off, CK)], didx)
            pltpu.async_copy(p_hbm.at[sidx], rows, sem).wait()
            pltpu.sync_copy(rows, acc.at[didx], add=True)
            return 0

        lax.fori_loop(0, NFULL, body, 0)
        if REM:
            off = base + NFULL * CK
            rs = sidx.at[pl.ds(0, REM)]
            rd = didx.at[pl.ds(0, REM)]
            rr = rows.at[pl.ds(0, REM)]
            pltpu.sync_copy(src_hbm.at[pl.ds(off, REM)], rs)
            pltpu.sync_copy(dst_hbm.at[pl.ds(off, REM)], rd)
            pltpu.async_copy(p_hbm.at[rs], rr, sem).wait()
            pltpu.sync_copy(rr, acc.at[rd], add=True)
        plsc.subcore_barrier()

        def outcopy(k, _):
            rk = r0 + k * ZR
            pltpu.sync_copy(acc.at[pl.ds(rk, ZR)], zbuf)
            pltpu.sync_copy(zbuf, out_hbm.at[cid, pl.ds(rk, ZR)])
            return 0

        lax.fori_loop(0, nk, outcopy, 0)

    return prop


def _deg_a(cnt):
    return lax.rsqrt(cnt[0, 0, :, 0] + cnt[0, 1, :, 0] + 1.0)


def _deg_b(cnt):
    return lax.rsqrt(cnt[1, 0, :, 0] + cnt[1, 1, :, 0] + 1.0)


def _mm1_body(x_ref, w_ref, cnt_ref, o_ref):
    a = _deg_a(cnt_ref[...])
    o_ref[...] = jnp.dot(x_ref[...], w_ref[...],
                         preferred_element_type=jnp.float32) * a[:, None]


def _mm2_body(t_ref, w_ref, cnt_ref, o_ref):
    cnt = cnt_ref[...]
    s = t_ref[0] + t_ref[1]
    h = jnp.maximum(s * _deg_b(cnt)[:, None], 0.0)
    o_ref[...] = jnp.dot(h, w_ref[...],
                         preferred_element_type=jnp.float32) * _deg_a(cnt)[:, None]


def _decode_body(tr_ref, cr_ref, tc_ref, cc_ref, o_ref):
    zr = (tr_ref[0] + tr_ref[1]) * _deg_b(cr_ref[...])[:, None]
    zc = (tc_ref[0] + tc_ref[1]) * _deg_b(cc_ref[...])[:, None]
    logits = lax.dot_general(zr, zc, (((1,), (1,)), ((), ())),
                             preferred_element_type=jnp.float32)
    o_ref[...] = 1.0 / (1.0 + jnp.exp(-logits))


def kernel(x, edge_index, W1, W2):
    N, F = x.shape
    E = edge_index.shape[1]
    H1 = W1.shape[1]
    H2 = W2.shape[1]
    src = edge_index[0]
    dst = edge_index[1]

    cnt = _make_deg_kernel(N, E)(src, dst)

    BR = 400
    nb = N // BR
    cnt_spec = pl.BlockSpec((2, 2, BR, 16), lambda i: (0, 0, i, 0))

    p1 = pl.pallas_call(
        _mm1_body,
        grid=(nb,),
        in_specs=[pl.BlockSpec((BR, F), lambda i: (i, 0)),
                  pl.BlockSpec((F, H1), lambda i: (0, 0)),
                  cnt_spec],
        out_specs=pl.BlockSpec((BR, H1), lambda i: (i, 0)),
        out_shape=jax.ShapeDtypeStruct((N, H1), jnp.float32),
    )(x, W1, cnt)

    t1 = _make_prop_kernel(N, E, H1)(p1, src, dst)

    p2 = pl.pallas_call(
        _mm2_body,
        grid=(nb,),
        in_specs=[pl.BlockSpec((NC, BR, H1), lambda i: (0, i, 0)),
                  pl.BlockSpec((H1, H2), lambda i: (0, 0)),
                  cnt_spec],
        out_specs=pl.BlockSpec((BR, H2), lambda i: (i, 0)),
        out_shape=jax.ShapeDtypeStruct((N, H2), jnp.float32),
    )(t1, W2, cnt)

    t2 = _make_prop_kernel(N, E, H2)(p2, src, dst)

    BC = 2048
    nc_blocks = pl.cdiv(N, BC)
    a_pred = pl.pallas_call(
        _decode_body,
        grid=(nb, nc_blocks),
        in_specs=[pl.BlockSpec((NC, BR, H2), lambda i, j: (0, i, 0)),
                  pl.BlockSpec((2, 2, BR, 16), lambda i, j: (0, 0, i, 0)),
                  pl.BlockSpec((NC, BC, H2), lambda i, j: (0, j, 0)),
                  pl.BlockSpec((2, 2, BC, 16), lambda i, j: (0, 0, j, 0))],
        out_specs=pl.BlockSpec((BR, BC), lambda i, j: (i, j)),
        out_shape=jax.ShapeDtypeStruct((N, N), jnp.float32),
    )(t2, cnt, t2, cnt)

    return a_pred
